# baseline (speedup 1.0000x reference)
import jax
import jax.numpy as jnp
from jax import lax
from jax.experimental import pallas as pl
from jax.experimental.pallas import tpu as pltpu

B = 32
N = 100000
A = 8
G = 4
BG = B // G
NB = 25600
NBLK = (N + NB - 1) // NB
NPAD = NBLK * NB
EXPLOIT = 0.9
NEG_INF = float("-inf")
BIG_I32 = 2**30


def _tc_body(qm_ref, qs_ref, eps_ref, at_ref,
             lp_ref, m_out_ref, ba_ref,
             u_sc, m_sc, s_sc, i_sc, idx_smem, win, sem, gsem):
    g = pl.program_id(0)
    ph = pl.program_id(1)
    j = pl.program_id(2)
    off = pl.multiple_of(j * NB, NB)

    def _update(um, lane):
        bm = jnp.max(um, axis=1, keepdims=True)
        bidx = jnp.min(jnp.where(um == bm, lane, BIG_I32), axis=1,
                       keepdims=True)

        @pl.when(j == 0)
        def _():
            m_sc[...] = bm
            s_sc[...] = jnp.sum(jnp.exp(um - bm), axis=1, keepdims=True)
            i_sc[...] = bidx

        @pl.when(j > 0)
        def _():
            m_old = m_sc[...]
            m_new = jnp.maximum(m_old, bm)
            s_sc[...] = (s_sc[...] * jnp.exp(m_old - m_new)
                         + jnp.sum(jnp.exp(um - m_new), axis=1,
                                   keepdims=True))
            i_sc[...] = jnp.where(bm > m_old, bidx, i_sc[...])
            m_sc[...] = m_new

    @pl.when(ph == 0)
    def _phase0():
        qs = qs_ref[...]
        u = EXPLOIT * (qm_ref[...] + qs * eps_ref[...]) + (1.0 - EXPLOIT) * qs
        u_sc[:, pl.ds(off, NB)] = u
        lane = lax.broadcasted_iota(jnp.int32, (BG, NB), 1) + off

        @pl.when(j < NBLK - 1)
        def _():
            _update(u, lane)

        @pl.when(j == NBLK - 1)
        def _():
            _update(jnp.where(lane < N, u, NEG_INF), lane)

        @pl.when(j == NBLK - 1)
        def _fire_gather():
            m_out_ref[...] = m_sc[...]
            pltpu.make_async_copy(i_sc, idx_smem, gsem).start()
            pltpu.make_async_copy(i_sc, idx_smem, gsem).wait()
            for b in range(BG):
                col0 = jnp.minimum((idx_smem[b, 0] // 128) * 128, N - 128)
                pltpu.make_async_copy(
                    at_ref.at[g * BG + b, :,
                              pl.ds(pl.multiple_of(col0, 128), 128)],
                    win.at[b], sem).start()

    @pl.when(ph == 1)
    def _phase1():
        lse = m_sc[...] + jnp.log(s_sc[...])
        lp_ref[...] = u_sc[:, pl.ds(off, NB)] - lse

        @pl.when(j == NBLK - 1)
        def _gather():
            for b in range(BG):
                col0 = jnp.minimum((idx_smem[b, 0] // 128) * 128, N - 128)
                pltpu.make_async_copy(
                    at_ref.at[g * BG + b, :,
                              pl.ds(pl.multiple_of(col0, 128), 128)],
                    win.at[b], sem).wait()
            idxv = i_sc[...].reshape(BG, 1, 1)
            cw = idxv - jnp.minimum((idxv // 128) * 128, N - 128)
            lane3 = lax.broadcasted_iota(jnp.int32, (BG, A, 128), 2)
            ba_ref[...] = jnp.sum(
                jnp.where(lane3 == cw, win[...], 0.0), axis=2)


def _tc_call(q_mean, q_std, eps, at):
    in_spec = pl.BlockSpec(
        (BG, NB), lambda g, ph, j: (g, jnp.where(ph == 0, j, 0)))
    return pl.pallas_call(
        _tc_body,
        grid=(G, 2, NBLK),
        in_specs=[
            in_spec, in_spec, in_spec,
            pl.BlockSpec(memory_space=pl.ANY),
        ],
        out_specs=[
            pl.BlockSpec((BG, NB),
                         lambda g, ph, j: (g, jnp.where(ph == 0, 0, j))),
            pl.BlockSpec((BG, 1), lambda g, ph, j: (g, 0)),
            pl.BlockSpec((BG, A), lambda g, ph, j: (g, 0)),
        ],
        out_shape=[
            jax.ShapeDtypeStruct((B, N), jnp.float32),
            jax.ShapeDtypeStruct((B, 1), jnp.float32),
            jax.ShapeDtypeStruct((B, A), jnp.float32),
        ],
        scratch_shapes=[
            pltpu.VMEM((BG, NPAD), jnp.float32),
            pltpu.VMEM((BG, 1), jnp.float32),
            pltpu.VMEM((BG, 1), jnp.float32),
            pltpu.VMEM((BG, 1), jnp.int32),
            pltpu.SMEM((BG, 1), jnp.int32),
            pltpu.VMEM((BG, A, 128), jnp.float32),
            pltpu.SemaphoreType.DMA,
            pltpu.SemaphoreType.DMA,
        ],
        compiler_params=pltpu.CompilerParams(
            dimension_semantics=("arbitrary", "arbitrary", "arbitrary")),
    )(q_mean, q_std, eps, at)


def kernel(q_mean, q_std, eps, action):
    at = action.transpose(0, 2, 1)
    log_probs, m, best_action = _tc_call(q_mean, q_std, eps, at)
    return log_probs, best_action, m.reshape(B)


# speedup vs baseline: 1.6096x; 1.6096x over previous
import jax
import jax.numpy as jnp
from jax import lax
from jax.experimental import pallas as pl
from jax.experimental.pallas import tpu as pltpu

B = 32
N = 100000
A = 8
G = 1
BG = B // G
NB = 25600
NBLK = (N + NB - 1) // NB
NPAD = NBLK * NB
EXPLOIT = 0.9
NEG_INF = float("-inf")
BIG_I32 = 2**30


def _tc_body(qm_ref, qs_ref, eps_ref, at_ref,
             lp_ref, m_out_ref, ba_ref,
             u_sc, m_sc, s_sc, i_sc, idx_smem, win, sem, gsem):
    g = pl.program_id(0)
    ph = pl.program_id(1)
    j = pl.program_id(2)
    off = pl.multiple_of(j * NB, NB)

    def _update(um, lane):
        bm = jnp.max(um, axis=1, keepdims=True)
        bidx = jnp.min(jnp.where(um == bm, lane, BIG_I32), axis=1,
                       keepdims=True)

        @pl.when(j == 0)
        def _():
            m_sc[...] = bm
            s_sc[...] = jnp.sum(jnp.exp(um - bm), axis=1, keepdims=True)
            i_sc[...] = bidx

        @pl.when(j > 0)
        def _():
            m_old = m_sc[...]
            m_new = jnp.maximum(m_old, bm)
            s_sc[...] = (s_sc[...] * jnp.exp(m_old - m_new)
                         + jnp.sum(jnp.exp(um - m_new), axis=1,
                                   keepdims=True))
            i_sc[...] = jnp.where(bm > m_old, bidx, i_sc[...])
            m_sc[...] = m_new

    @pl.when(ph == 0)
    def _phase0():
        qs = qs_ref[...]
        u = EXPLOIT * (qm_ref[...] + qs * eps_ref[...]) + (1.0 - EXPLOIT) * qs
        u_sc[:, pl.ds(off, NB)] = u
        lane = lax.broadcasted_iota(jnp.int32, (BG, NB), 1) + off

        @pl.when(j < NBLK - 1)
        def _():
            _update(u, lane)

        @pl.when(j == NBLK - 1)
        def _():
            _update(jnp.where(lane < N, u, NEG_INF), lane)

        @pl.when(j == NBLK - 1)
        def _fire_gather():
            m_out_ref[...] = m_sc[...]
            pltpu.make_async_copy(i_sc, idx_smem, gsem).start()
            pltpu.make_async_copy(i_sc, idx_smem, gsem).wait()
            for b in range(BG):
                col0 = jnp.minimum((idx_smem[b, 0] // 128) * 128, N - 128)
                pltpu.make_async_copy(
                    at_ref.at[g * BG + b, :,
                              pl.ds(pl.multiple_of(col0, 128), 128)],
                    win.at[b], sem).start()

    @pl.when(ph == 1)
    def _phase1():
        lse = m_sc[...] + jnp.log(s_sc[...])
        lp_ref[...] = u_sc[:, pl.ds(off, NB)] - lse

        @pl.when(j == NBLK - 1)
        def _gather():
            for b in range(BG):
                col0 = jnp.minimum((idx_smem[b, 0] // 128) * 128, N - 128)
                pltpu.make_async_copy(
                    at_ref.at[g * BG + b, :,
                              pl.ds(pl.multiple_of(col0, 128), 128)],
                    win.at[b], sem).wait()
            idxv = i_sc[...].reshape(BG, 1, 1)
            cw = idxv - jnp.minimum((idxv // 128) * 128, N - 128)
            lane3 = lax.broadcasted_iota(jnp.int32, (BG, A, 128), 2)
            ba_ref[...] = jnp.sum(
                jnp.where(lane3 == cw, win[...], 0.0), axis=2)


def _tc_call(q_mean, q_std, eps, at):
    in_spec = pl.BlockSpec(
        (BG, NB), lambda g, ph, j: (g, jnp.where(ph == 0, j, 0)))
    return pl.pallas_call(
        _tc_body,
        grid=(G, 2, NBLK),
        in_specs=[
            in_spec, in_spec, in_spec,
            pl.BlockSpec(memory_space=pl.ANY),
        ],
        out_specs=[
            pl.BlockSpec((BG, NB),
                         lambda g, ph, j: (g, jnp.where(ph == 0, 0, j))),
            pl.BlockSpec((BG, 1), lambda g, ph, j: (g, 0)),
            pl.BlockSpec((BG, A), lambda g, ph, j: (g, 0)),
        ],
        out_shape=[
            jax.ShapeDtypeStruct((B, N), jnp.float32),
            jax.ShapeDtypeStruct((B, 1), jnp.float32),
            jax.ShapeDtypeStruct((B, A), jnp.float32),
        ],
        scratch_shapes=[
            pltpu.VMEM((BG, NPAD), jnp.float32),
            pltpu.VMEM((BG, 1), jnp.float32),
            pltpu.VMEM((BG, 1), jnp.float32),
            pltpu.VMEM((BG, 1), jnp.int32),
            pltpu.SMEM((BG, 1), jnp.int32),
            pltpu.VMEM((BG, A, 128), jnp.float32),
            pltpu.SemaphoreType.DMA,
            pltpu.SemaphoreType.DMA,
        ],
        compiler_params=pltpu.CompilerParams(
            dimension_semantics=("arbitrary", "arbitrary", "arbitrary")),
    )(q_mean, q_std, eps, at)


def kernel(q_mean, q_std, eps, action):
    at = action.transpose(0, 2, 1)
    log_probs, m, best_action = _tc_call(q_mean, q_std, eps, at)
    return log_probs, best_action, m.reshape(B)
